# baseline (speedup 1.0000x reference)
import dataclasses
import functools

import jax
import jax.numpy as jnp
from jax import lax
from jax.experimental import pallas as pl
from jax.experimental.pallas import tpu as pltpu
from jax.experimental.pallas import tpu_sc as plsc

N = 10000
D = 128
E = 320000

NC = 2
NS = 16
NW = NC * NS
EW = E // NW
CH = 40
NCHUNK = EW // CH
CPI = 5
L = 16

CC = CPI * CH
NP = 10240
HR = NP // L
CHB = 2000
NCHB = EW // CHB

_cp = pltpu.CompilerParams()
if "needs_layout_passes" in pltpu.CompilerParams.__dataclass_fields__:
    _cp = dataclasses.replace(_cp, needs_layout_passes=False)

_mesh = plsc.VectorSubcoreMesh(core_axis_name="c", subcore_axis_name="s")


def _normalize_body(x_ref, o_ref):
    xb = x_ref[...]
    nrm = jnp.sqrt(jnp.sum(xb * xb, axis=1, keepdims=True))
    o_ref[...] = xb / jnp.maximum(nrm, 1e-12)


def _normalize(x):
    return pl.pallas_call(
        _normalize_body,
        out_shape=jax.ShapeDtypeStruct((N, D), jnp.float32),
        grid=(10,),
        in_specs=[pl.BlockSpec((N // 10, D), lambda i: (i, 0))],
        out_specs=pl.BlockSpec((N // 10, D), lambda i: (i, 0)),
    )(x)


@functools.partial(
    pl.kernel,
    out_type=(
        jax.ShapeDtypeStruct((E,), jnp.float32),
        jax.ShapeDtypeStruct((NC, HR, L), jnp.float32),
    ),
    mesh=_mesh,
    compiler_params=_cp,
    scratch_types=[
        pltpu.VMEM((2 * CC,), jnp.int32),
        pltpu.VMEM((CC,), jnp.float32),
        pltpu.VMEM((CH, D), jnp.float32),
        pltpu.VMEM((CH, D), jnp.float32),
        pltpu.VMEM((CH, D), jnp.float32),
        pltpu.VMEM((CH, D), jnp.float32),
        pltpu.VMEM((CH, D), jnp.float32),
        pltpu.VMEM((CH, D), jnp.float32),
        pltpu.VMEM((HR, L), jnp.float32),
        pltpu.VMEM((5, 128), jnp.int32),
        pltpu.VMEM_SHARED((HR, L), jnp.float32),
        pltpu.SemaphoreType.DMA,
        pltpu.SemaphoreType.DMA,
        pltpu.SemaphoreType.DMA,
        pltpu.SemaphoreType.DMA,
    ],
)
def _edge_kernel(xn_hbm, eidx_hbm, zeros_hbm, idx5_hbm, sim_hbm,
                 part_hbm, eit, simt, a0, a1, a2, b0, b1, b2,
                 priv_v, idxm_v, hist_sh, sem0, sem1, sem2, sem_w):
    cid = lax.axis_index("c")
    sid = lax.axis_index("s")
    wid = cid * NS + sid
    base = wid * EW
    lane = lax.iota(jnp.int32, L)

    av = (a0, a1, a2)
    bv = (b0, b1, b2)
    sems = (sem0, sem1, sem2)

    @pl.when(sid == 0)
    def _():
        pltpu.sync_copy(zeros_hbm, hist_sh)

    pltpu.sync_copy(zeros_hbm, priv_v)
    pltpu.sync_copy(idx5_hbm, idxm_v)

    @pl.loop(0, NCHUNK // CPI)
    def _iter(it):
        off0 = base + it * CC
        pltpu.sync_copy(eidx_hbm.at[wid * (NCHUNK // CPI) + it], eit)

        @pl.when(it > 0)
        def _():
            pltpu.make_async_copy(simt, sim_hbm.at[pl.ds(base, CC)],
                                  sem_w).wait()
        handles = {}

        def issue(j):
            b = j % 3
            ha = pltpu.async_copy(xn_hbm.at[eit.at[pl.ds(j * CH, CH)]],
                                  av[b], sems[b])
            hb = pltpu.async_copy(xn_hbm.at[eit.at[pl.ds(CC + j * CH, CH)]],
                                  bv[b], sems[b])
            handles[j] = (ha, hb)

        issue(0)
        issue(1)
        for j in range(CPI):
            b = j % 3
            ha, hb = handles[j]
            ha.wait()
            hb.wait()
            for g in range(CH // L):
                r16 = eit[pl.ds(j * CH + g * L, L)]
                c16 = eit[pl.ds(CC + j * CH + g * L, L)]
                s16 = jnp.zeros((L,), jnp.float32)
                for e in range(L):
                    acc = av[b][g * L + e, pl.ds(0, L)] * bv[b][g * L + e, pl.ds(0, L)]
                    for k in range(1, D // L):
                        acc = acc + (av[b][g * L + e, pl.ds(k * L, L)]
                                     * bv[b][g * L + e, pl.ds(k * L, L)])
                    s16 = jnp.where(lane == e, jnp.sum(acc), s16)
                s16 = jnp.where(s16 < 0.1, 0.0, s16)
                s16 = jnp.where(r16 == c16, 0.0, s16)
                simt[pl.ds(j * CH + g * L, L)] = s16
                plsc.addupdate_scatter(
                    priv_v,
                    [lax.shift_right_logical(r16, 4), lax.bitwise_and(r16, 15)],
                    s16)
            if j + 2 < CPI:
                issue(j + 2)
        pltpu.async_copy(simt, sim_hbm.at[pl.ds(off0, CC)], sem_w)

    pltpu.make_async_copy(simt, sim_hbm.at[pl.ds(base, CC)], sem_w).wait()

    plsc.subcore_barrier()
    for j in range(5):
        pltpu.sync_copy(priv_v.at[pl.ds(j * 128, 128)],
                        hist_sh.at[idxm_v.at[j]], add=True)
    plsc.subcore_barrier()

    @pl.when(sid == 0)
    def _():
        pltpu.sync_copy(hist_sh, part_hbm.at[cid])


@functools.partial(
    pl.kernel,
    out_type=jax.ShapeDtypeStruct((E,), jnp.float32),
    mesh=_mesh,
    compiler_params=_cp,
    scratch_types=[
        pltpu.VMEM((NP,), jnp.float32),
        pltpu.VMEM((NP,), jnp.float32),
        pltpu.VMEM((CHB,), jnp.int32),
        pltpu.VMEM((CHB,), jnp.int32),
        pltpu.VMEM((CHB,), jnp.float32),
        pltpu.VMEM((CHB,), jnp.float32),
        pltpu.VMEM((CHB,), jnp.float32),
        pltpu.VMEM((CHB,), jnp.float32),
        pltpu.SemaphoreType.DMA,
        pltpu.SemaphoreType.DMA,
        pltpu.SemaphoreType.DMA,
        pltpu.SemaphoreType.DMA,
    ],
)
def _div_kernel(sim_hbm, row_hbm, part_hbm, out_hbm, rs0, rs1,
                ridx0, ridx1, sv0, sv1, ov0, ov1,
                sem_l0, sem_l1, sem_w0, sem_w1):
    cid = lax.axis_index("c")
    sid = lax.axis_index("s")
    wid = cid * NS + sid
    base = wid * EW

    ridx = (ridx0, ridx1)
    sv = (sv0, sv1)
    ov = (ov0, ov1)
    sem_l = (sem_l0, sem_l1)
    sem_w = (sem_w0, sem_w1)

    load_handles = {}

    def issue_load(c):
        p = c % 2
        off = base + c * CHB
        h1 = pltpu.async_copy(row_hbm.at[pl.ds(off, CHB)], ridx[p], sem_l[p])
        h2 = pltpu.async_copy(sim_hbm.at[pl.ds(off, CHB)], sv[p], sem_l[p])
        load_handles[c] = (h1, h2)

    issue_load(0)
    pltpu.sync_copy(part_hbm.at[0], rs0)
    pltpu.sync_copy(part_hbm.at[1], rs1)
    issue_load(1)

    @pl.loop(0, HR)
    def _acc(i):
        rs0[pl.ds(i * L, L)] = rs0[pl.ds(i * L, L)] + rs1[pl.ds(i * L, L)]

    write_handles = {}
    for c in range(NCHB):
        p = c % 2
        h1, h2 = load_handles[c]
        h1.wait()
        h2.wait()
        if c - 2 >= 0:
            w1 = write_handles[c - 2]
            w1.wait()
        off = base + c * CHB

        @pl.loop(0, CHB // L)
        def _grp(g):
            idx = ridx[p][pl.ds(g * L, L)]
            den = plsc.load_gather(rs0, [idx])
            s16 = sv[p][pl.ds(g * L, L)]
            ov[p][pl.ds(g * L, L)] = s16 / jnp.maximum(den, 1e-12)

        if c + 2 < NCHB:
            issue_load(c + 2)
        write_handles[c] = pltpu.async_copy(ov[p], out_hbm.at[pl.ds(off, CHB)],
                                            sem_w[p])
    for c in range(max(0, NCHB - 2), NCHB):
        write_handles[c].wait()


def kernel(x, edge_index):
    row = edge_index[0]
    niter = NCHUNK // CPI
    eidx = (edge_index.reshape(2, NW, niter, CC)
            .transpose(1, 2, 0, 3).reshape(NW * niter, 2 * CC))
    xn = _normalize(x)
    zeros = jnp.zeros((HR, L), jnp.float32)
    idx5 = jnp.arange(HR, dtype=jnp.int32).reshape(5, 128)
    sim, part = _edge_kernel(xn, eidx, zeros, idx5)
    return _div_kernel(sim, row, part.reshape(NC, NP))


# speedup vs baseline: 1.0227x; 1.0227x over previous
import dataclasses
import functools

import jax
import jax.numpy as jnp
from jax import lax
from jax.experimental import pallas as pl
from jax.experimental.pallas import tpu as pltpu
from jax.experimental.pallas import tpu_sc as plsc

N = 10000
D = 128
E = 320000

NC = 2
NS = 16
NW = NC * NS
EW = E // NW
CH = 40
NCHUNK = EW // CH
CPI = 5
L = 16

CC = CPI * CH
NP = 10240
HR = NP // L
CHB = 2000
NCHB = EW // CHB

_cp = pltpu.CompilerParams()
if "needs_layout_passes" in pltpu.CompilerParams.__dataclass_fields__:
    _cp = dataclasses.replace(_cp, needs_layout_passes=False)

_mesh = plsc.VectorSubcoreMesh(core_axis_name="c", subcore_axis_name="s")


def _normalize_body(x_ref, o_ref):
    xb = x_ref[...]
    nrm = jnp.sqrt(jnp.sum(xb * xb, axis=1, keepdims=True))
    o_ref[...] = xb / jnp.maximum(nrm, 1e-12)


def _normalize(x):
    return pl.pallas_call(
        _normalize_body,
        out_shape=jax.ShapeDtypeStruct((N, D), jnp.float32),
        grid=(10,),
        in_specs=[pl.BlockSpec((N // 10, D), lambda i: (i, 0))],
        out_specs=pl.BlockSpec((N // 10, D), lambda i: (i, 0)),
    )(x)


@functools.partial(
    pl.kernel,
    out_type=(
        jax.ShapeDtypeStruct((E,), jnp.float32),
        jax.ShapeDtypeStruct((NC, HR, L), jnp.float32),
    ),
    mesh=_mesh,
    compiler_params=_cp,
    scratch_types=[
        pltpu.VMEM((2 * CC,), jnp.int32),
        pltpu.VMEM((CC,), jnp.float32),
        pltpu.VMEM((CH, D), jnp.float32),
        pltpu.VMEM((CH, D), jnp.float32),
        pltpu.VMEM((CH, D), jnp.float32),
        pltpu.VMEM((CH, D), jnp.float32),
        pltpu.VMEM((CH, D), jnp.float32),
        pltpu.VMEM((CH, D), jnp.float32),
        pltpu.VMEM((HR, L), jnp.float32),
        pltpu.VMEM((5, 128), jnp.int32),
        pltpu.VMEM_SHARED((HR, L), jnp.float32),
        pltpu.SemaphoreType.DMA,
        pltpu.SemaphoreType.DMA,
        pltpu.SemaphoreType.DMA,
        pltpu.SemaphoreType.DMA,
    ],
)
def _edge_kernel(xn_hbm, eidx_hbm, zeros_hbm, idx5_hbm, sim_hbm,
                 part_hbm, eit, simt, a0, a1, a2, b0, b1, b2,
                 priv_v, idxm_v, hist_sh, sem0, sem1, sem2, sem_w):
    cid = lax.axis_index("c")
    sid = lax.axis_index("s")
    wid = cid * NS + sid
    base = wid * EW
    lane = lax.iota(jnp.int32, L)

    av = (a0, a1, a2)
    bv = (b0, b1, b2)
    sems = (sem0, sem1, sem2)

    @pl.when(sid == 0)
    def _():
        pltpu.sync_copy(zeros_hbm, hist_sh)

    pltpu.sync_copy(zeros_hbm, priv_v)
    pltpu.sync_copy(idx5_hbm, idxm_v)

    @pl.loop(0, NCHUNK // CPI)
    def _iter(it):
        off0 = base + it * CC
        pltpu.sync_copy(eidx_hbm.at[wid * (NCHUNK // CPI) + it], eit)

        @pl.when(it > 0)
        def _():
            pltpu.make_async_copy(simt, sim_hbm.at[pl.ds(base, CC)],
                                  sem_w).wait()
        handles = {}

        def issue(j):
            b = j % 3
            ha = pltpu.async_copy(xn_hbm.at[eit.at[pl.ds(j * CH, CH)]],
                                  av[b], sems[b])
            hb = pltpu.async_copy(xn_hbm.at[eit.at[pl.ds(CC + j * CH, CH)]],
                                  bv[b], sems[b])
            handles[j] = (ha, hb)

        issue(0)
        issue(1)
        for j in range(CPI):
            b = j % 3
            ha, hb = handles[j]
            ha.wait()
            hb.wait()
            if j + 2 < CPI:
                issue(j + 2)
            for g in range(CH // L):
                r16 = eit[pl.ds(j * CH + g * L, L)]
                c16 = eit[pl.ds(CC + j * CH + g * L, L)]
                s16 = jnp.zeros((L,), jnp.float32)
                for e in range(L):
                    acc = av[b][g * L + e, pl.ds(0, L)] * bv[b][g * L + e, pl.ds(0, L)]
                    for k in range(1, D // L):
                        acc = acc + (av[b][g * L + e, pl.ds(k * L, L)]
                                     * bv[b][g * L + e, pl.ds(k * L, L)])
                    s16 = jnp.where(lane == e, jnp.sum(acc), s16)
                s16 = jnp.where(s16 < 0.1, 0.0, s16)
                s16 = jnp.where(r16 == c16, 0.0, s16)
                simt[pl.ds(j * CH + g * L, L)] = s16
                plsc.addupdate_scatter(
                    priv_v,
                    [lax.shift_right_logical(r16, 4), lax.bitwise_and(r16, 15)],
                    s16)
        pltpu.async_copy(simt, sim_hbm.at[pl.ds(off0, CC)], sem_w)

    pltpu.make_async_copy(simt, sim_hbm.at[pl.ds(base, CC)], sem_w).wait()

    plsc.subcore_barrier()
    for j in range(5):
        pltpu.sync_copy(priv_v.at[pl.ds(j * 128, 128)],
                        hist_sh.at[idxm_v.at[j]], add=True)
    plsc.subcore_barrier()

    @pl.when(sid == 0)
    def _():
        pltpu.sync_copy(hist_sh, part_hbm.at[cid])


@functools.partial(
    pl.kernel,
    out_type=jax.ShapeDtypeStruct((E,), jnp.float32),
    mesh=_mesh,
    compiler_params=_cp,
    scratch_types=[
        pltpu.VMEM((NP,), jnp.float32),
        pltpu.VMEM((NP,), jnp.float32),
        pltpu.VMEM((CHB,), jnp.int32),
        pltpu.VMEM((CHB,), jnp.int32),
        pltpu.VMEM((CHB,), jnp.float32),
        pltpu.VMEM((CHB,), jnp.float32),
        pltpu.VMEM((CHB,), jnp.float32),
        pltpu.VMEM((CHB,), jnp.float32),
        pltpu.SemaphoreType.DMA,
        pltpu.SemaphoreType.DMA,
        pltpu.SemaphoreType.DMA,
        pltpu.SemaphoreType.DMA,
    ],
)
def _div_kernel(sim_hbm, row_hbm, part_hbm, out_hbm, rs0, rs1,
                ridx0, ridx1, sv0, sv1, ov0, ov1,
                sem_l0, sem_l1, sem_w0, sem_w1):
    cid = lax.axis_index("c")
    sid = lax.axis_index("s")
    wid = cid * NS + sid
    base = wid * EW

    ridx = (ridx0, ridx1)
    sv = (sv0, sv1)
    ov = (ov0, ov1)
    sem_l = (sem_l0, sem_l1)
    sem_w = (sem_w0, sem_w1)

    load_handles = {}

    def issue_load(c):
        p = c % 2
        off = base + c * CHB
        h1 = pltpu.async_copy(row_hbm.at[pl.ds(off, CHB)], ridx[p], sem_l[p])
        h2 = pltpu.async_copy(sim_hbm.at[pl.ds(off, CHB)], sv[p], sem_l[p])
        load_handles[c] = (h1, h2)

    issue_load(0)
    pltpu.sync_copy(part_hbm.at[0], rs0)
    pltpu.sync_copy(part_hbm.at[1], rs1)
    issue_load(1)

    @pl.loop(0, HR)
    def _acc(i):
        rs0[pl.ds(i * L, L)] = rs0[pl.ds(i * L, L)] + rs1[pl.ds(i * L, L)]

    write_handles = {}
    for c in range(NCHB):
        p = c % 2
        h1, h2 = load_handles[c]
        h1.wait()
        h2.wait()
        if c - 2 >= 0:
            w1 = write_handles[c - 2]
            w1.wait()
        off = base + c * CHB

        @pl.loop(0, CHB // L)
        def _grp(g):
            idx = ridx[p][pl.ds(g * L, L)]
            den = plsc.load_gather(rs0, [idx])
            s16 = sv[p][pl.ds(g * L, L)]
            ov[p][pl.ds(g * L, L)] = s16 / jnp.maximum(den, 1e-12)

        if c + 2 < NCHB:
            issue_load(c + 2)
        write_handles[c] = pltpu.async_copy(ov[p], out_hbm.at[pl.ds(off, CHB)],
                                            sem_w[p])
    for c in range(max(0, NCHB - 2), NCHB):
        write_handles[c].wait()


def kernel(x, edge_index):
    row = edge_index[0]
    niter = NCHUNK // CPI
    eidx = (edge_index.reshape(2, NW, niter, CC)
            .transpose(1, 2, 0, 3).reshape(NW * niter, 2 * CC))
    xn = _normalize(x)
    zeros = jnp.zeros((HR, L), jnp.float32)
    idx5 = jnp.arange(HR, dtype=jnp.int32).reshape(5, 128)
    sim, part = _edge_kernel(xn, eidx, zeros, idx5)
    return _div_kernel(sim, row, part.reshape(NC, NP))
